# baseline (speedup 1.0000x reference)
import functools
import math

import jax
import jax.numpy as jnp
from jax import lax
from jax.experimental import pallas as pl
from jax.experimental.pallas import tpu as pltpu
from jax.experimental.pallas import tpu_sc as plsc

N = 10000
H = 256
NC, NS, L = 2, 16, 16
NW = NC * NS
NV = 10240
SINK = N
AR = 10240
ECH = 256
EPT = 5120
E_PAD = NW * EPT
NCH = EPT // ECH
K0 = 40
EPT_MAX = K0 * ECH
NCHA = 16 * K0
E_ALLOC = NCHA * ECH
RT = 400
RS = 1.0 / math.sqrt(1.0 + 1e-5)

@functools.cache
def _mesh():
    return plsc.VectorSubcoreMesh(core_axis_name="c", subcore_axis_name="s",
                                  num_cores=NC, num_subcores=NS)


def _pre_body(x_ref, win_ref, bin_ref, gin_ref, bbn_ref, wgat_ref, wgcn_ref,
              asrc_ref, adst_ref, h_ref, hg_ref, hc_ref, ssrc_ref, sdst_ref):
    y = jnp.dot(x_ref[...], win_ref[...], preferred_element_type=jnp.float32)
    h = jnp.maximum(gin_ref[...] * RS * (y + bin_ref[...]) + bbn_ref[...], 0.0)
    hg = jnp.dot(h, wgat_ref[...], preferred_element_type=jnp.float32)
    hc = jnp.dot(h, wgcn_ref[...], preferred_element_type=jnp.float32)
    h_ref[...] = h
    hg_ref[...] = hg
    hc_ref[...] = hc
    ssrc_ref[...] = jnp.dot(hg, asrc_ref[...], preferred_element_type=jnp.float32)
    sdst_ref[...] = jnp.dot(hg, adst_ref[...], preferred_element_type=jnp.float32)


def _pre(x, p):
    full = lambda shape: pl.BlockSpec(shape, lambda i: (0, 0))
    row = pl.BlockSpec((RT, H), lambda i: (i, 0))
    col = pl.BlockSpec((RT, 1), lambda i: (i, 0))
    return pl.pallas_call(
        _pre_body,
        grid=(N // RT,),
        in_specs=[row, full((H, H)), full((1, H)), full((1, H)), full((1, H)),
                  full((H, H)), full((H, H)), full((H, 1)), full((H, 1))],
        out_specs=[row, row, row, col, col],
        out_shape=[jax.ShapeDtypeStruct((N, H), jnp.float32)] * 3
        + [jax.ShapeDtypeStruct((N, 1), jnp.float32)] * 2,
    )(x, p['W_in'], p['b_in'].reshape(1, H), p['bn_in_g'].reshape(1, H),
      p['bn_in_b'].reshape(1, H), p['W_gat'], p['W_gcn'],
      p['a_src'].reshape(H, 1), p['a_dst'].reshape(H, 1))


@functools.cache
def _edge_scalar_kernel():
    return functools.partial(
        pl.kernel,
        out_type=[jax.ShapeDtypeStruct((E_ALLOC,), jnp.float32),
                  jax.ShapeDtypeStruct((NC * NV,), jnp.float32),
                  jax.ShapeDtypeStruct((NC * NV,), jnp.float32)],
        mesh=_mesh(),
        scratch_types=[pltpu.VMEM((NV,), jnp.float32),
                       pltpu.VMEM((NV,), jnp.float32),
                       pltpu.VMEM((EPT,), jnp.int32),
                       pltpu.VMEM((EPT,), jnp.int32),
                       pltpu.VMEM((EPT,), jnp.float32),
                       pltpu.VMEM((NV,), jnp.float32),
                       pltpu.VMEM((NV,), jnp.float32),
                       pltpu.VMEM((NS, NV // NS), jnp.float32),
                       pltpu.VMEM((NV // NS,), jnp.float32),
                       pltpu.VMEM_SHARED((NS, NV), jnp.float32)],
        compiler_params=pltpu.CompilerParams(
            use_tc_tiling_on_sc=False, needs_layout_passes=False),
    )(_edge_scalar)


def _edge_scalar(ssrc_hbm, sdst_hbm, src_hbm, dst_hbm,
                 wg_hbm, degp_hbm, denp_hbm,
                 ssrc_v, sdst_v, src_v, dst_v, w_v, deg_v, den_v,
                 red_v, accred_v, sh):
    cid = lax.axis_index("c")
    sid = lax.axis_index("s")
    wid = sid * NC + cid
    ebase = wid * EPT
    pltpu.sync_copy(ssrc_hbm, ssrc_v)
    pltpu.sync_copy(sdst_hbm, sdst_v)
    pltpu.sync_copy(src_hbm.at[pl.ds(ebase, EPT)], src_v)
    pltpu.sync_copy(dst_hbm.at[pl.ds(ebase, EPT)], dst_v)

    zero = jnp.zeros((L,), jnp.float32)

    def z_body(i, _):
        deg_v[pl.ds(i * L, L)] = zero
        den_v[pl.ds(i * L, L)] = zero
        return 0

    lax.fori_loop(0, NV // L, z_body, 0)

    ones = jnp.ones((L,), jnp.float32)

    def e_body(i, _):
        s16 = src_v[pl.ds(i * L, L)]
        d16 = dst_v[pl.ds(i * L, L)]
        a = plsc.load_gather(ssrc_v, [s16]) + plsc.load_gather(sdst_v, [d16])
        a = jnp.maximum(a, 0.2 * a)
        w = jnp.exp(a)
        w_v[pl.ds(i * L, L)] = w
        plsc.addupdate_scatter(den_v, [d16], w)
        plsc.addupdate_scatter(deg_v, [d16], ones)
        return 0

    lax.fori_loop(0, EPT // L, e_body, 0)

    pltpu.sync_copy(w_v, wg_hbm.at[pl.ds(ebase, EPT)])

    sl = NV // NS
    myoff = sid * sl

    def _reduce(part_v, out_hbm):
        pltpu.sync_copy(part_v, sh.at[sid])
        plsc.subcore_barrier()
        for t in range(NS):
            pltpu.sync_copy(sh.at[t, pl.ds(myoff, sl)], red_v.at[t])

        def r_body(j, _):
            acc = red_v[0, pl.ds(j * L, L)]
            for t in range(1, NS):
                acc = acc + red_v[t, pl.ds(j * L, L)]
            accred_v[pl.ds(j * L, L)] = acc
            return 0

        lax.fori_loop(0, sl // L, r_body, 0)
        pltpu.sync_copy(accred_v, out_hbm.at[pl.ds(cid * NV + myoff, sl)])
        plsc.subcore_barrier()

    _reduce(deg_v, degp_hbm)
    _reduce(den_v, denp_hbm)


@functools.cache
def _edge_vector_kernel():
    return functools.partial(
        pl.kernel,
        out_type=jax.ShapeDtypeStruct((12 * N, 64), jnp.float32),
        mesh=_mesh(),
        scratch_types=[pltpu.VMEM((EPT_MAX,), jnp.int32),
                       pltpu.VMEM((K0, ECH), jnp.int32),
                       pltpu.VMEM((EPT_MAX,), jnp.float32),
                       pltpu.VMEM((K0, ECH), jnp.int32),
                       pltpu.VMEM((ECH, 64), jnp.float32),
                       pltpu.VMEM((ECH, 64), jnp.float32),
                       pltpu.VMEM((64, 64), jnp.float32),
                       pltpu.VMEM_SHARED((AR, 64), jnp.float32),
                       pltpu.SemaphoreType.DMA,
                       pltpu.SemaphoreType.DMA],
        compiler_params=pltpu.CompilerParams(
            use_tc_tiling_on_sc=False, needs_layout_passes=False),
    )(_edge_vector)


def _edge_vector(t_hbm, src_hbm, dst2_hbm, wgat_hbm,
                 p_hbm, src_v, dstc_v, wg_v, idxq_v, rows0_v, rows1_v,
                 zero_v, acc, gsem0, gsem1):
    cid = lax.axis_index("c")
    sid = lax.axis_index("s")
    k_ch = K0
    cbase = sid * K0
    ebase = cbase * ECH

    @pl.when(cid == 0)
    def _loads():
        pltpu.sync_copy(src_hbm.at[pl.ds(ebase, EPT_MAX)], src_v)
        pltpu.sync_copy(dst2_hbm.at[pl.ds(cbase, K0)], dstc_v)
        pltpu.sync_copy(wgat_hbm.at[pl.ds(ebase, EPT_MAX)], wg_v)

    zero = jnp.zeros((L,), jnp.float32)

    def zv_body(i, _):
        zero_v[i // 4, pl.ds((i % 4) * L, L)] = zero
        return 0

    lax.fori_loop(0, 64 * 4, zv_body, 0)

    zrows = AR // NS
    frows = N // NS
    bufs = ((rows0_v, gsem0), (rows1_v, gsem1))

    def pass_body(pidx, _):
        g = pidx // 4
        off = pidx % 4 + g * (4 * N)

        def ix_body(ci, _):
            for k in range(ECH // L):
                s16 = src_v[pl.ds(ci * ECH + k * L, L)] * 4
                idxq_v[ci, pl.ds(k * L, L)] = s16 + off
            return 0

        lax.fori_loop(0, k_ch, ix_body, 0)
        pltpu.async_copy(t_hbm.at[idxq_v.at[0]], rows0_v, gsem0)
        pltpu.async_copy(t_hbm.at[idxq_v.at[1]], rows1_v, gsem1)
        with jax.named_scope("ev_zero"):
            for z in range(zrows // 64):
                pltpu.sync_copy(zero_v,
                                acc.at[pl.ds(sid * zrows + z * 64, 64)])
            plsc.subcore_barrier()

        def pair_body(i, _):
            for p, (rows, gsem) in enumerate(bufs):
                ci = i * 2 + p
                pltpu.make_async_copy(
                    t_hbm.at[idxq_v.at[0]], rows, gsem).wait()

                @pl.when(g == 0)
                def _scale():
                    def sc_body(r, _):
                        wv = plsc.load_gather(
                            wg_v, [jnp.full((L,), ci * ECH + r, jnp.int32)])
                        for q in range(64 // L):
                            rows[r, pl.ds(q * L, L)] = (
                                rows[r, pl.ds(q * L, L)] * wv)
                        return 0

                    lax.fori_loop(0, ECH, sc_body, 0)

                pltpu.sync_copy(rows, acc.at[dstc_v.at[ci]], add=True)

                @pl.when(ci + 2 < k_ch)
                def _issue():
                    pltpu.async_copy(t_hbm.at[idxq_v.at[ci + 2]], rows, gsem)
            return 0

        with jax.named_scope("ev_edges"):
            lax.fori_loop(0, k_ch // 2, pair_body, 0)
            plsc.subcore_barrier()
        with jax.named_scope("ev_flush"):
            fbase = sid * frows
            pltpu.sync_copy(
                acc.at[pl.ds(fbase, frows)],
                p_hbm.at[pl.ds(pidx * N + fbase, frows)])
            plsc.subcore_barrier()
        return 0

    @pl.when(cid == 0)
    def _run():
        lax.fori_loop(0, 12, pass_body, 0)


def _scale_body(hc_ref, dinv_ref, out_ref):
    out_ref[...] = dinv_ref[...] * hc_ref[...]


def _scale_hc(hc, dinv):
    row = pl.BlockSpec((RT, H), lambda i: (i, 0))
    col = pl.BlockSpec((RT, 1), lambda i: (i, 0))
    return pl.pallas_call(
        _scale_body,
        grid=(N // RT,),
        in_specs=[row, col],
        out_specs=row,
        out_shape=jax.ShapeDtypeStruct((N, H), jnp.float32),
    )(hc, dinv)


def _elu(v):
    return jnp.where(v > 0.0, v, jnp.exp(jnp.minimum(v, 0.0)) - 1.0)


def _post_body(p_ref, h_ref, hg_ref, hc_ref, ssrc_ref, sdst_ref, deg_ref,
               den_ref, wsl_ref, wsr_ref, bsage_ref, bgat_ref, bgcn_ref,
               wr1_ref, br1_ref, gr1_ref, hr1_ref, wr2_ref, br2_ref, gr2_ref,
               hr2_ref, wc1_ref, bc1_ref, gc_ref, hcb_ref, wc2_ref, bc2_ref,
               wc3_ref, bc3_ref, temp_ref, out_ref):
    a = p_ref[...]
    a_gat = jnp.concatenate([a[0], a[1], a[2], a[3]], axis=1)
    a_gcn = jnp.concatenate([a[4], a[5], a[6], a[7]], axis=1)
    a_sag = jnp.concatenate([a[8], a[9], a[10], a[11]], axis=1)
    h = h_ref[...]
    hg = hg_ref[...]
    hcs = hc_ref[...]
    s = ssrc_ref[...] + sdst_ref[...]
    exself = jnp.exp(jnp.maximum(s, 0.2 * s))
    denom = den_ref[...] + exself
    x1 = _elu((a_gat + exself * hg) / (denom + 1e-16) + bgat_ref[...])
    deg = deg_ref[...]
    dinv = lax.rsqrt(deg + 1.0)
    x2 = _elu(dinv * (a_gcn + hcs) + bgcn_ref[...])
    agg = a_sag / jnp.maximum(deg, 1.0)
    x3 = _elu(jnp.dot(agg, wsl_ref[...], preferred_element_type=jnp.float32)
              + bsage_ref[...]
              + jnp.dot(h, wsr_ref[...], preferred_element_type=jnp.float32))
    h2 = x1 + x2 + x3 + h
    r = jnp.dot(h2, wr1_ref[...], preferred_element_type=jnp.float32)
    r = jnp.maximum(gr1_ref[...] * RS * (r + br1_ref[...]) + hr1_ref[...], 0.0)
    r = jnp.dot(r, wr2_ref[...], preferred_element_type=jnp.float32)
    r = gr2_ref[...] * RS * (r + br2_ref[...]) + hr2_ref[...]
    h3 = jnp.maximum(r + h2, 0.0)
    c = jnp.dot(h3, wc1_ref[...], preferred_element_type=jnp.float32)
    c = gc_ref[...] * RS * jnp.maximum(c + bc1_ref[...], 0.0) + hcb_ref[...]
    c = jnp.dot(c, wc2_ref[...], preferred_element_type=jnp.float32)
    c = jnp.maximum(c + bc2_ref[...], 0.0)
    c = jnp.dot(c, wc3_ref[...], preferred_element_type=jnp.float32)
    out_ref[...] = (c + bc3_ref[...]) / temp_ref[...]


def _post(pacc, h, hg, hc, ssrc, sdst, deg, den, p):
    full = lambda shape: pl.BlockSpec(shape, lambda i: tuple(0 for _ in shape))
    row = pl.BlockSpec((RT, H), lambda i: (i, 0))
    col = pl.BlockSpec((RT, 1), lambda i: (i, 0))
    pspec = pl.BlockSpec((12, RT, 64), lambda i: (0, i, 0))
    return pl.pallas_call(
        _post_body,
        grid=(N // RT,),
        in_specs=[pspec, row, row, row, col, col, col, col,
                  full((H, H)), full((H, H)), full((1, H)), full((1, H)),
                  full((1, H)),
                  full((H, H)), full((1, H)), full((1, H)), full((1, H)),
                  full((H, H)), full((1, H)), full((1, H)), full((1, H)),
                  full((H, 128)), full((1, 128)), full((1, 128)),
                  full((1, 128)),
                  full((128, 64)), full((1, 64)), full((64, 1)), full((1, 1)),
                  full((1, 1))],
        out_specs=col,
        out_shape=jax.ShapeDtypeStruct((N, 1), jnp.float32),
    )(pacc, h, hg, hc, ssrc, sdst, deg, den,
      p['W_sage_l'], p['W_sage_r'], p['b_sage'].reshape(1, H),
      p['b_gat'].reshape(1, H), p['b_gcn'].reshape(1, H),
      p['W_r1'], p['b_r1'].reshape(1, H), p['bn_r1_g'].reshape(1, H),
      p['bn_r1_b'].reshape(1, H),
      p['W_r2'], p['b_r2'].reshape(1, H), p['bn_r2_g'].reshape(1, H),
      p['bn_r2_b'].reshape(1, H),
      p['W_c1'], p['b_c1'].reshape(1, 128), p['bn_c_g'].reshape(1, 128),
      p['bn_c_b'].reshape(1, 128),
      p['W_c2'], p['b_c2'].reshape(1, 64), p['W_c3'],
      p['b_c3'].reshape(1, 1), p['temperature'].reshape(1, 1))


def kernel(x, edge_index, params):
    E = edge_index.shape[1]
    h, hg, hc, ssrc, sdst = _pre(x, params)

    src = edge_index[0].astype(jnp.int32)
    dst = edge_index[1].astype(jnp.int32)
    srcp = jnp.concatenate([src, jnp.zeros((E_ALLOC - E,), jnp.int32)])
    dstp = jnp.concatenate([dst, jnp.full((E_ALLOC - E,), SINK, jnp.int32)])
    zpad = jnp.zeros((NV - N,), jnp.float32)
    ssrc_f = jnp.concatenate([ssrc[:, 0], zpad])
    sdst_f = jnp.concatenate([sdst[:, 0], zpad])

    wgat, degp, denp = _edge_scalar_kernel()(ssrc_f, sdst_f, srcp, dstp)
    deg = degp[:NV] + degp[NV:]
    den = denp[:NV] + denp[NV:]
    deg2d = deg[:N].reshape(N, 1)
    hcs = _scale_hc(hc, lax.rsqrt(deg2d + 1.0))

    t = jnp.concatenate([hg, hcs, h], axis=0).reshape(12 * N, 64)
    pacc = _edge_vector_kernel()(t, srcp, dstp.reshape(NCHA, ECH), wgat)

    out = _post(pacc.reshape(12, N, 64), h, hg, hcs, ssrc, sdst,
                deg2d, den[:N].reshape(N, 1), params)
    return out[:, 0]


# speedup vs baseline: 1.2757x; 1.2757x over previous
import functools
import math

import jax
import jax.numpy as jnp
from jax import lax
from jax.experimental import pallas as pl
from jax.experimental.pallas import tpu as pltpu
from jax.experimental.pallas import tpu_sc as plsc

N = 10000
H = 256
NC, NS, L = 2, 16, 16
NW = NC * NS
NV = 10240
SINK = N
AR = 10240
ECH = 256
EPT = 5120
E_PAD = NW * EPT
NCH = EPT // ECH
K0 = 30
K1 = 10
EPT_MAX = K0 * ECH
NCHA = 16 * (K0 + K1) + 2 * L
E_ALLOC = NCHA * ECH
RT = 400
RS = 1.0 / math.sqrt(1.0 + 1e-5)

@functools.cache
def _mesh():
    return plsc.VectorSubcoreMesh(core_axis_name="c", subcore_axis_name="s",
                                  num_cores=NC, num_subcores=NS)


def _pre_body(x_ref, win_ref, bin_ref, gin_ref, bbn_ref, wgat_ref, wgcn_ref,
              asrc_ref, adst_ref, h_ref, hg_ref, hc_ref, ssrc_ref, sdst_ref):
    y = jnp.dot(x_ref[...], win_ref[...], preferred_element_type=jnp.float32)
    h = jnp.maximum(gin_ref[...] * RS * (y + bin_ref[...]) + bbn_ref[...], 0.0)
    hg = jnp.dot(h, wgat_ref[...], preferred_element_type=jnp.float32)
    hc = jnp.dot(h, wgcn_ref[...], preferred_element_type=jnp.float32)
    h_ref[...] = h
    hg_ref[...] = hg
    hc_ref[...] = hc
    ssrc_ref[...] = jnp.dot(hg, asrc_ref[...], preferred_element_type=jnp.float32)
    sdst_ref[...] = jnp.dot(hg, adst_ref[...], preferred_element_type=jnp.float32)


def _pre(x, p):
    full = lambda shape: pl.BlockSpec(shape, lambda i: (0, 0))
    row = pl.BlockSpec((RT, H), lambda i: (i, 0))
    col = pl.BlockSpec((RT, 1), lambda i: (i, 0))
    return pl.pallas_call(
        _pre_body,
        grid=(N // RT,),
        in_specs=[row, full((H, H)), full((1, H)), full((1, H)), full((1, H)),
                  full((H, H)), full((H, H)), full((H, 1)), full((H, 1))],
        out_specs=[row, row, row, col, col],
        out_shape=[jax.ShapeDtypeStruct((N, H), jnp.float32)] * 3
        + [jax.ShapeDtypeStruct((N, 1), jnp.float32)] * 2,
    )(x, p['W_in'], p['b_in'].reshape(1, H), p['bn_in_g'].reshape(1, H),
      p['bn_in_b'].reshape(1, H), p['W_gat'], p['W_gcn'],
      p['a_src'].reshape(H, 1), p['a_dst'].reshape(H, 1))


@functools.cache
def _edge_scalar_kernel():
    return functools.partial(
        pl.kernel,
        out_type=[jax.ShapeDtypeStruct((E_ALLOC,), jnp.float32),
                  jax.ShapeDtypeStruct((NC * NV,), jnp.float32),
                  jax.ShapeDtypeStruct((NC * NV,), jnp.float32)],
        mesh=_mesh(),
        scratch_types=[pltpu.VMEM((NV,), jnp.float32),
                       pltpu.VMEM((NV,), jnp.float32),
                       pltpu.VMEM((EPT,), jnp.int32),
                       pltpu.VMEM((EPT,), jnp.int32),
                       pltpu.VMEM((EPT,), jnp.float32),
                       pltpu.VMEM((NV,), jnp.float32),
                       pltpu.VMEM((NV,), jnp.float32),
                       pltpu.VMEM((NS, NV // NS), jnp.float32),
                       pltpu.VMEM((NV // NS,), jnp.float32),
                       pltpu.VMEM_SHARED((NS, NV), jnp.float32)],
        compiler_params=pltpu.CompilerParams(
            use_tc_tiling_on_sc=False, needs_layout_passes=False),
    )(_edge_scalar)


def _edge_scalar(ssrc_hbm, sdst_hbm, src_hbm, dst_hbm,
                 wg_hbm, degp_hbm, denp_hbm,
                 ssrc_v, sdst_v, src_v, dst_v, w_v, deg_v, den_v,
                 red_v, accred_v, sh):
    cid = lax.axis_index("c")
    sid = lax.axis_index("s")
    wid = sid * NC + cid
    ebase = wid * EPT
    pltpu.sync_copy(ssrc_hbm, ssrc_v)
    pltpu.sync_copy(sdst_hbm, sdst_v)
    pltpu.sync_copy(src_hbm.at[pl.ds(ebase, EPT)], src_v)
    pltpu.sync_copy(dst_hbm.at[pl.ds(ebase, EPT)], dst_v)

    zero = jnp.zeros((L,), jnp.float32)

    def z_body(i, _):
        deg_v[pl.ds(i * L, L)] = zero
        den_v[pl.ds(i * L, L)] = zero
        return 0

    lax.fori_loop(0, NV // L, z_body, 0)

    ones = jnp.ones((L,), jnp.float32)

    def e_body(i, _):
        s16 = src_v[pl.ds(i * L, L)]
        d16 = dst_v[pl.ds(i * L, L)]
        a = plsc.load_gather(ssrc_v, [s16]) + plsc.load_gather(sdst_v, [d16])
        a = jnp.maximum(a, 0.2 * a)
        w = jnp.exp(a)
        w_v[pl.ds(i * L, L)] = w
        plsc.addupdate_scatter(den_v, [d16], w)
        plsc.addupdate_scatter(deg_v, [d16], ones)
        return 0

    lax.fori_loop(0, EPT // L, e_body, 0)

    pltpu.sync_copy(w_v, wg_hbm.at[pl.ds(ebase, EPT)])

    sl = NV // NS
    myoff = sid * sl

    def _reduce(part_v, out_hbm):
        pltpu.sync_copy(part_v, sh.at[sid])
        plsc.subcore_barrier()
        for t in range(NS):
            pltpu.sync_copy(sh.at[t, pl.ds(myoff, sl)], red_v.at[t])

        def r_body(j, _):
            acc = red_v[0, pl.ds(j * L, L)]
            for t in range(1, NS):
                acc = acc + red_v[t, pl.ds(j * L, L)]
            accred_v[pl.ds(j * L, L)] = acc
            return 0

        lax.fori_loop(0, sl // L, r_body, 0)
        pltpu.sync_copy(accred_v, out_hbm.at[pl.ds(cid * NV + myoff, sl)])
        plsc.subcore_barrier()

    _reduce(deg_v, degp_hbm)
    _reduce(den_v, denp_hbm)


@functools.cache
def _edge_vector_kernel():
    return functools.partial(
        pl.kernel,
        out_type=jax.ShapeDtypeStruct((NC * 12 * N, 64), jnp.float32),
        mesh=_mesh(),
        scratch_types=[pltpu.VMEM((EPT_MAX,), jnp.int32),
                       pltpu.VMEM((K0, ECH), jnp.int32),
                       pltpu.VMEM((EPT_MAX,), jnp.float32),
                       pltpu.VMEM((K0, ECH), jnp.int32),
                       pltpu.VMEM((ECH, 64), jnp.float32),
                       pltpu.VMEM((ECH, 64), jnp.float32),
                       pltpu.VMEM((64, 64), jnp.float32),
                       pltpu.VMEM_SHARED((AR, 64), jnp.float32),
                       pltpu.SemaphoreType.DMA,
                       pltpu.SemaphoreType.DMA],
        compiler_params=pltpu.CompilerParams(
            use_tc_tiling_on_sc=False, needs_layout_passes=False),
    )(_edge_vector)


def _edge_vector(hg_hbm, hcs_hbm, h_hbm, src_hbm, dst2_hbm, wgat_hbm,
                 p_hbm, src_v, dstc_v, wg_v, idxq_v, rows0_v, rows1_v,
                 zero_v, acc, gsem0, gsem1):
    cid = lax.axis_index("c")
    sid = lax.axis_index("s")
    k_ch = jnp.where(cid == 0, K0, K1)
    cbase = jnp.where(cid == 0, sid * K0, 16 * K0 + sid * K1)
    ebase = cbase * ECH
    pltpu.sync_copy(src_hbm.at[pl.ds(ebase, EPT_MAX)], src_v)
    pltpu.sync_copy(dst2_hbm.at[pl.ds(cbase, K0)], dstc_v)
    pltpu.sync_copy(wgat_hbm.at[pl.ds(ebase, EPT_MAX)], wg_v)

    zero = jnp.zeros((L,), jnp.float32)

    def zv_body(i, _):
        zero_v[i // 4, pl.ds((i % 4) * L, L)] = zero
        return 0

    lax.fori_loop(0, 64 * 4, zv_body, 0)

    zrows = AR // NS
    frows = N // NS
    bufs = ((rows0_v, gsem0), (rows1_v, gsem1))

    for jq in range(4):
        def ix_body(ci, _):
            for k in range(ECH // L):
                s16 = src_v[pl.ds(ci * ECH + k * L, L)] * 4
                idxq_v[ci, pl.ds(k * L, L)] = s16 + jq
            return 0

        lax.fori_loop(0, k_ch, ix_body, 0)
        for g in range(3):
            table = (hg_hbm, hcs_hbm, h_hbm)[g]
            pidx = g * 4 + jq
            pltpu.async_copy(table.at[idxq_v.at[0]], rows0_v, gsem0)
            pltpu.async_copy(table.at[idxq_v.at[1]], rows1_v, gsem1)
            for z in range(zrows // 64):
                pltpu.sync_copy(zero_v, acc.at[pl.ds(sid * zrows + z * 64, 64)])
            plsc.subcore_barrier()

            def pair_body(i, _):
                for p, (rows, gsem) in enumerate(bufs):
                    ci = i * 2 + p
                    pltpu.make_async_copy(
                        table.at[idxq_v.at[0]], rows, gsem).wait()
                    if g == 0:
                        def sc_body(r, _):
                            wv = plsc.load_gather(
                                wg_v,
                                [jnp.full((L,), ci * ECH + r, jnp.int32)])
                            for q in range(64 // L):
                                rows[r, pl.ds(q * L, L)] = (
                                    rows[r, pl.ds(q * L, L)] * wv)
                            return 0

                        lax.fori_loop(0, ECH, sc_body, 0)
                    pltpu.sync_copy(rows, acc.at[dstc_v.at[ci]], add=True)

                    @pl.when(ci + 2 < k_ch)
                    def _issue():
                        pltpu.async_copy(
                            table.at[idxq_v.at[ci + 2]], rows, gsem)
                return 0

            lax.fori_loop(0, k_ch // 2, pair_body, 0)
            plsc.subcore_barrier()
            fbase = sid * frows
            pltpu.sync_copy(
                acc.at[pl.ds(fbase, frows)],
                p_hbm.at[pl.ds((cid * 12 + pidx) * N + fbase, frows)])
            plsc.subcore_barrier()


def _scale_body(hc_ref, dinv_ref, out_ref):
    out_ref[...] = dinv_ref[...] * hc_ref[...]


def _scale_hc(hc, dinv):
    row = pl.BlockSpec((RT, H), lambda i: (i, 0))
    col = pl.BlockSpec((RT, 1), lambda i: (i, 0))
    return pl.pallas_call(
        _scale_body,
        grid=(N // RT,),
        in_specs=[row, col],
        out_specs=row,
        out_shape=jax.ShapeDtypeStruct((N, H), jnp.float32),
    )(hc, dinv)


def _elu(v):
    return jnp.where(v > 0.0, v, jnp.exp(jnp.minimum(v, 0.0)) - 1.0)


def _post_body(p_ref, h_ref, hg_ref, hc_ref, ssrc_ref, sdst_ref, deg_ref,
               den_ref, wsl_ref, wsr_ref, bsage_ref, bgat_ref, bgcn_ref,
               wr1_ref, br1_ref, gr1_ref, hr1_ref, wr2_ref, br2_ref, gr2_ref,
               hr2_ref, wc1_ref, bc1_ref, gc_ref, hcb_ref, wc2_ref, bc2_ref,
               wc3_ref, bc3_ref, temp_ref, out_ref):
    pp = p_ref[...]
    a = pp[0] + pp[1]
    a_gat = jnp.concatenate([a[0], a[1], a[2], a[3]], axis=1)
    a_gcn = jnp.concatenate([a[4], a[5], a[6], a[7]], axis=1)
    a_sag = jnp.concatenate([a[8], a[9], a[10], a[11]], axis=1)
    h = h_ref[...]
    hg = hg_ref[...]
    hcs = hc_ref[...]
    s = ssrc_ref[...] + sdst_ref[...]
    exself = jnp.exp(jnp.maximum(s, 0.2 * s))
    denom = den_ref[...] + exself
    x1 = _elu((a_gat + exself * hg) / (denom + 1e-16) + bgat_ref[...])
    deg = deg_ref[...]
    dinv = lax.rsqrt(deg + 1.0)
    x2 = _elu(dinv * (a_gcn + hcs) + bgcn_ref[...])
    agg = a_sag / jnp.maximum(deg, 1.0)
    x3 = _elu(jnp.dot(agg, wsl_ref[...], preferred_element_type=jnp.float32)
              + bsage_ref[...]
              + jnp.dot(h, wsr_ref[...], preferred_element_type=jnp.float32))
    h2 = x1 + x2 + x3 + h
    r = jnp.dot(h2, wr1_ref[...], preferred_element_type=jnp.float32)
    r = jnp.maximum(gr1_ref[...] * RS * (r + br1_ref[...]) + hr1_ref[...], 0.0)
    r = jnp.dot(r, wr2_ref[...], preferred_element_type=jnp.float32)
    r = gr2_ref[...] * RS * (r + br2_ref[...]) + hr2_ref[...]
    h3 = jnp.maximum(r + h2, 0.0)
    c = jnp.dot(h3, wc1_ref[...], preferred_element_type=jnp.float32)
    c = gc_ref[...] * RS * jnp.maximum(c + bc1_ref[...], 0.0) + hcb_ref[...]
    c = jnp.dot(c, wc2_ref[...], preferred_element_type=jnp.float32)
    c = jnp.maximum(c + bc2_ref[...], 0.0)
    c = jnp.dot(c, wc3_ref[...], preferred_element_type=jnp.float32)
    out_ref[...] = (c + bc3_ref[...]) / temp_ref[...]


def _post(pacc, h, hg, hc, ssrc, sdst, deg, den, p):
    full = lambda shape: pl.BlockSpec(shape, lambda i: tuple(0 for _ in shape))
    row = pl.BlockSpec((RT, H), lambda i: (i, 0))
    col = pl.BlockSpec((RT, 1), lambda i: (i, 0))
    pspec = pl.BlockSpec((NC, 12, RT, 64), lambda i: (0, 0, i, 0))
    return pl.pallas_call(
        _post_body,
        grid=(N // RT,),
        in_specs=[pspec, row, row, row, col, col, col, col,
                  full((H, H)), full((H, H)), full((1, H)), full((1, H)),
                  full((1, H)),
                  full((H, H)), full((1, H)), full((1, H)), full((1, H)),
                  full((H, H)), full((1, H)), full((1, H)), full((1, H)),
                  full((H, 128)), full((1, 128)), full((1, 128)),
                  full((1, 128)),
                  full((128, 64)), full((1, 64)), full((64, 1)), full((1, 1)),
                  full((1, 1))],
        out_specs=col,
        out_shape=jax.ShapeDtypeStruct((N, 1), jnp.float32),
    )(pacc, h, hg, hc, ssrc, sdst, deg, den,
      p['W_sage_l'], p['W_sage_r'], p['b_sage'].reshape(1, H),
      p['b_gat'].reshape(1, H), p['b_gcn'].reshape(1, H),
      p['W_r1'], p['b_r1'].reshape(1, H), p['bn_r1_g'].reshape(1, H),
      p['bn_r1_b'].reshape(1, H),
      p['W_r2'], p['b_r2'].reshape(1, H), p['bn_r2_g'].reshape(1, H),
      p['bn_r2_b'].reshape(1, H),
      p['W_c1'], p['b_c1'].reshape(1, 128), p['bn_c_g'].reshape(1, 128),
      p['bn_c_b'].reshape(1, 128),
      p['W_c2'], p['b_c2'].reshape(1, 64), p['W_c3'],
      p['b_c3'].reshape(1, 1), p['temperature'].reshape(1, 1))


def kernel(x, edge_index, params):
    E = edge_index.shape[1]
    h, hg, hc, ssrc, sdst = _pre(x, params)

    src = edge_index[0].astype(jnp.int32)
    dst = edge_index[1].astype(jnp.int32)
    srcp = jnp.concatenate([src, jnp.zeros((E_ALLOC - E,), jnp.int32)])
    dstp = jnp.concatenate([dst, jnp.full((E_ALLOC - E,), SINK, jnp.int32)])
    zpad = jnp.zeros((NV - N,), jnp.float32)
    ssrc_f = jnp.concatenate([ssrc[:, 0], zpad])
    sdst_f = jnp.concatenate([sdst[:, 0], zpad])

    wgat, degp, denp = _edge_scalar_kernel()(ssrc_f, sdst_f, srcp, dstp)
    deg = degp[:NV] + degp[NV:]
    den = denp[:NV] + denp[NV:]
    deg2d = deg[:N].reshape(N, 1)
    hcs = _scale_hc(hc, lax.rsqrt(deg2d + 1.0))

    pacc = _edge_vector_kernel()(hg.reshape(4 * N, 64), hcs.reshape(4 * N, 64),
                                 h.reshape(4 * N, 64), srcp,
                                 dstp.reshape(NCHA, ECH), wgat)

    out = _post(pacc.reshape(NC, 12, N, 64), h, hg, hcs, ssrc, sdst,
                deg2d, den[:N].reshape(N, 1), params)
    return out[:, 0]
